# baseline (speedup 1.0000x reference)
import functools

import jax
import jax.numpy as jnp
from jax import lax
from jax.experimental import pallas as pl
from jax.experimental.pallas import tpu as pltpu
from jax.experimental.pallas import tpu_sc as plsc

EMB_DIM = 64
BATCH = 16384
NUM_CORES = 2
NUM_SUBCORES = 16
NUM_WORKERS = NUM_CORES * NUM_SUBCORES
B_PER_W = BATCH // NUM_WORKERS
CHUNK = 128
N_CHUNKS = B_PER_W // CHUNK


def _gather_body(user_idx, item_idx, rating_idx, user_emb, item_emb,
                 opinion_emb, p_out, q_out, e_out,
                 uidx_v, iidx_v, ridx_v, urows_v, irows_v, rrows_v,
                 idx_sem, gat_sem, out_sem):
    wid = lax.axis_index("s") * NUM_CORES + lax.axis_index("c")
    row0 = wid * N_CHUNKS
    base = wid * B_PER_W

    cp_u = pltpu.async_copy(user_idx.at[pl.ds(row0, N_CHUNKS)], uidx_v, idx_sem)
    cp_i = pltpu.async_copy(item_idx.at[pl.ds(row0, N_CHUNKS)], iidx_v, idx_sem)
    cp_r = pltpu.async_copy(rating_idx.at[pl.ds(row0, N_CHUNKS)], ridx_v, idx_sem)
    cp_u.wait()
    cp_i.wait()
    cp_r.wait()

    gathers = []
    for table, idx_v, rows_v in ((user_emb, uidx_v, urows_v),
                                 (item_emb, iidx_v, irows_v),
                                 (opinion_emb, ridx_v, rrows_v)):
        for j in range(N_CHUNKS):
            gathers.append(pltpu.async_copy(
                table.at[idx_v.at[j]],
                rows_v.at[pl.ds(j * CHUNK, CHUNK)],
                gat_sem))

    outs = []
    for t, (rows_v, out) in enumerate(((urows_v, p_out), (irows_v, q_out),
                                       (rrows_v, e_out))):
        for j in range(N_CHUNKS):
            gathers[t * N_CHUNKS + j].wait()
        outs.append(pltpu.async_copy(rows_v, out.at[pl.ds(base, B_PER_W)],
                                     out_sem))
    for cp in outs:
        cp.wait()


@jax.jit
def _run(user_idx, item_idx, rating_idx, user_emb, item_emb, opinion_emb):
    mesh = plsc.VectorSubcoreMesh(core_axis_name="c", subcore_axis_name="s",
                                  num_cores=NUM_CORES,
                                  num_subcores=NUM_SUBCORES)
    out = jax.ShapeDtypeStruct((BATCH, EMB_DIM), jnp.float32)
    f = pl.kernel(
        _gather_body,
        out_type=(out, out, out),
        mesh=mesh,
        scratch_types=[
            pltpu.VMEM((N_CHUNKS, CHUNK), jnp.int32),
            pltpu.VMEM((N_CHUNKS, CHUNK), jnp.int32),
            pltpu.VMEM((N_CHUNKS, CHUNK), jnp.int32),
            pltpu.VMEM((B_PER_W, EMB_DIM), jnp.float32),
            pltpu.VMEM((B_PER_W, EMB_DIM), jnp.float32),
            pltpu.VMEM((B_PER_W, EMB_DIM), jnp.float32),
            pltpu.SemaphoreType.DMA,
            pltpu.SemaphoreType.DMA,
            pltpu.SemaphoreType.DMA,
        ],
        compiler_params=pltpu.CompilerParams(use_tc_tiling_on_sc=False),
    )
    return f(user_idx, item_idx, rating_idx, user_emb, item_emb, opinion_emb)


def kernel(user_idx, item_idx, rating_idx, user_emb, item_emb, opinion_emb):
    u = jnp.reshape(user_idx.astype(jnp.int32), (BATCH // CHUNK, CHUNK))
    i = jnp.reshape(item_idx.astype(jnp.int32), (BATCH // CHUNK, CHUNK))
    r = jnp.reshape(rating_idx.astype(jnp.int32), (BATCH // CHUNK, CHUNK))
    return _run(u, i, r, user_emb, item_emb, opinion_emb)


# speedup vs baseline: 1.4638x; 1.4638x over previous
import jax
import jax.numpy as jnp
from jax import lax
from jax.experimental import pallas as pl
from jax.experimental.pallas import tpu as pltpu
from jax.experimental.pallas import tpu_sc as plsc

EMB_DIM = 64
BATCH = 16384
NUM_CORES = 2
NUM_SUBCORES = 16
NUM_WORKERS = NUM_CORES * NUM_SUBCORES
B_PER_W = BATCH // NUM_WORKERS
CHUNK = 128
N_CHUNKS = B_PER_W // CHUNK
GROUPS = CHUNK // 16


def _gather_one(tab, dummy, ix_v, buf, out, base, gsem, osem):
    out_cps = [None] * N_CHUNKS
    for c in range(N_CHUNKS):
        if c >= 2:
            out_cps[c - 2].wait()
        b = buf.at[c % 2]

        def step(g, _, c=c, b=b):
            v = ix_v[pl.ds(c * CHUNK + g * 16, 16)]
            for k in range(16):
                pltpu.async_copy(tab.at[pl.ds(v[k], 1)],
                                 b.at[pl.ds(g * 16 + k, 1)], gsem)
            return 0

        lax.fori_loop(0, GROUPS, step, 0)
        pltpu.make_async_copy(dummy, b, gsem).wait()
        out_cps[c] = pltpu.async_copy(
            b, out.at[pl.ds(base + c * CHUNK, CHUNK)], osem)
    out_cps[N_CHUNKS - 2].wait()
    out_cps[N_CHUNKS - 1].wait()


def _body(uidx, iidx, ridx, utab, itab, rtab, p_out, q_out, e_out,
          uix_v, iix_v, rix_v, ubuf, ibuf, rbuf, gsem, osem):
    wid = lax.axis_index("s") * NUM_CORES + lax.axis_index("c")
    base = wid * B_PER_W
    pltpu.sync_copy(uidx.at[pl.ds(base, B_PER_W)], uix_v)
    pltpu.sync_copy(iidx.at[pl.ds(base, B_PER_W)], iix_v)
    pltpu.sync_copy(ridx.at[pl.ds(base, B_PER_W)], rix_v)

    dummy = utab.at[pl.ds(0, CHUNK)]
    _gather_one(utab, dummy, uix_v, ubuf, p_out, base, gsem, osem)
    _gather_one(itab, dummy, iix_v, ibuf, q_out, base, gsem, osem)
    _gather_one(rtab, dummy, rix_v, rbuf, e_out, base, gsem, osem)


@jax.jit
def _run(user_idx, item_idx, rating_idx, user_emb, item_emb, opinion_emb):
    mesh = plsc.VectorSubcoreMesh(core_axis_name="c", subcore_axis_name="s",
                                  num_cores=NUM_CORES,
                                  num_subcores=NUM_SUBCORES)
    out = jax.ShapeDtypeStruct((BATCH, EMB_DIM), jnp.float32)
    f = pl.kernel(
        _body,
        out_type=(out, out, out),
        mesh=mesh,
        scratch_types=[
            pltpu.VMEM((B_PER_W,), jnp.int32),
            pltpu.VMEM((B_PER_W,), jnp.int32),
            pltpu.VMEM((B_PER_W,), jnp.int32),
            pltpu.VMEM((2, CHUNK, EMB_DIM), jnp.float32),
            pltpu.VMEM((2, CHUNK, EMB_DIM), jnp.float32),
            pltpu.VMEM((2, CHUNK, EMB_DIM), jnp.float32),
            pltpu.SemaphoreType.DMA,
            pltpu.SemaphoreType.DMA,
        ],
    )
    return f(user_idx, item_idx, rating_idx, user_emb, item_emb, opinion_emb)


def kernel(user_idx, item_idx, rating_idx, user_emb, item_emb, opinion_emb):
    return _run(user_idx.astype(jnp.int32), item_idx.astype(jnp.int32),
                rating_idx.astype(jnp.int32), user_emb, item_emb, opinion_emb)


# speedup vs baseline: 2.9103x; 1.9882x over previous
import jax
import jax.numpy as jnp
from jax import lax
from jax.experimental import pallas as pl
from jax.experimental.pallas import tpu as pltpu
from jax.experimental.pallas import tpu_sc as plsc

EMB_DIM = 64
BATCH = 16384
N_ROWS = 1000000
NUM_CORES = 2
NUM_SUBCORES = 16
NUM_WORKERS = NUM_CORES * NUM_SUBCORES
B_PER_W = BATCH // NUM_WORKERS
PR = 512
N_PANELS = N_ROWS // PR
V_MAIN = N_PANELS * PR
PAN_BASE = N_PANELS // NUM_WORKERS
CAP = 2048
MAX_ROUNDS = (BATCH + CAP - 1) // CAP
PANEL_BYTES = EMB_DIM * PR * 4
ROW_BYTES = EMB_DIM * 4


def _stream_gather(tabT, tail_tab, out, idxbuf, lv, lp, pv, pp, pbuf, rowbuf,
                   wid, psem, osem, tsem):
    iota = lax.iota(jnp.int32, 16)
    p0 = PAN_BASE * wid + jnp.minimum(wid, 1)
    npan = jnp.where(wid == 0, PAN_BASE + 1, PAN_BASE)
    pend = p0 + npan

    def count_step(i, n_total):
        v = idxbuf[pl.ds(i * 16, 16)]
        pan = v >> 9
        m = (pan >= p0) & (pan < pend)
        n_total += plsc.all_reduce_population_count(m)[0]
        t = v >= V_MAIN

        ti = t.astype(jnp.int32)

        @pl.when((wid == NUM_WORKERS - 1)
                 & (plsc.all_reduce_population_count(t)[0] > 0))
        def _():
            for k in range(16):
                @pl.when(ti[k] != 0)
                def _():
                    s = v[k] - V_MAIN
                    pltpu.async_copy(tail_tab.at[pl.ds(s, 1)],
                                     rowbuf.at[0].at[pl.ds(0, 1)], tsem).wait()
                    pltpu.async_copy(rowbuf.at[0].at[pl.ds(0, 1)],
                                     out.at[pl.ds(i * 16 + k, 1)], tsem).wait()
        return n_total

    n_total = lax.fori_loop(0, BATCH // 16, count_step, 0)

    def round_body(r, _):
        @pl.when(r * CAP < n_total)
        def _():
            def scan_step(i, c):
                cg, cs = c
                v = idxbuf[pl.ds(i * 16, 16)]
                pos = iota + i * 16
                pan = v >> 9
                m = (pan >= p0) & (pan < pend)
                mi = m.astype(jnp.int32)
                ordv = cg + plsc.cumsum(mi) - mi
                sm = m & (ordv >= r * CAP) & (ordv < (r + 1) * CAP)
                plsc.store_compressed(lv.at[pl.ds(cs, 16)], v, mask=sm)
                plsc.store_compressed(lp.at[pl.ds(cs, 16)], pos, mask=sm)
                cg += plsc.all_reduce_population_count(m)[0]
                cs += plsc.all_reduce_population_count(sm)[0]
                return (cg, cs)

            _, n_list = lax.fori_loop(0, BATCH // 16, scan_step, (0, 0))
            nb = (n_list + 15) >> 4

            pltpu.async_copy(
                tabT.at[:, pl.ds(pl.multiple_of(p0 * PR, PR), PR)],
                pbuf.at[0], psem)

            def panel_step(j, gcnt):
                pan_id = p0 + j
                pltpu.make_async_copy(tabT.at[:, pl.ds(0, PR)],
                                      pbuf.at[j & 1], psem).wait()

                @pl.when(j + 1 < npan)
                def _():
                    pltpu.async_copy(
                        tabT.at[:, pl.ds(
                            pl.multiple_of((pan_id + 1) * PR, PR), PR)],
                        pbuf.at[(j + 1) & 1], psem)

                def mini(b, mc):
                    vv = lv[pl.ds(b * 16, 16)]
                    qq = lp[pl.ds(b * 16, 16)]
                    hit = ((iota + b * 16) < n_list) & ((vv >> 9) == pan_id)
                    plsc.store_compressed(pv.at[pl.ds(mc, 16)], vv, mask=hit)
                    plsc.store_compressed(pp.at[pl.ds(mc, 16)], qq, mask=hit)
                    return mc + plsc.all_reduce_population_count(hit)[0]

                mcnt = lax.fori_loop(0, nb, mini, 0)
                ng = (mcnt + 15) >> 4

                def group(g, gc):
                    par = gc & 1

                    @pl.when(gc >= 2)
                    def _():
                        pltpu.make_async_copy(out.at[pl.ds(0, 16)],
                                              rowbuf.at[par], osem).wait()

                    vv = pv[pl.ds(g * 16, 16)]
                    qq = pp[pl.ds(g * 16, 16)]
                    mval = (iota + g * 16) < mcnt
                    vv = jnp.where(mval, vv, jnp.broadcast_to(vv[0], (16,)))
                    qq = jnp.where(mval, qq, jnp.broadcast_to(qq[0], (16,)))
                    rr = vv & (PR - 1)
                    rb = rowbuf.at[par]
                    for c in range(EMB_DIM):
                        cc = jnp.full((16,), c, jnp.int32)
                        col = plsc.load_gather(pbuf.at[j & 1], [cc, rr])
                        plsc.store_scatter(rb, [iota, cc], col)
                    for k in range(16):
                        pltpu.async_copy(rb.at[pl.ds(k, 1)],
                                         out.at[pl.ds(qq[k], 1)], osem)
                    return gc + 1

                return lax.fori_loop(0, ng, group, gcnt)

            gcnt = lax.fori_loop(0, npan, panel_step, 0)

            @pl.when(gcnt >= 1)
            def _():
                pltpu.make_async_copy(out.at[pl.ds(0, 16)],
                                      rowbuf.at[(gcnt - 1) & 1], osem).wait()

            @pl.when(gcnt >= 2)
            def _():
                pltpu.make_async_copy(out.at[pl.ds(0, 16)],
                                      rowbuf.at[gcnt & 1], osem).wait()
        return 0

    lax.fori_loop(0, MAX_ROUNDS, round_body, 0)


def _body(uidx, iidx, ridx, utabT, itabT, rtab, utail, itail,
          p_out, q_out, e_out,
          idxbuf, ridxv, rtv, lv, lp, pv, pp, pbuf, rowbuf, rrow,
          psem, osem, tsem, rsem):
    wid = lax.axis_index("s") * NUM_CORES + lax.axis_index("c")
    base = wid * B_PER_W
    iota = lax.iota(jnp.int32, 16)

    pltpu.sync_copy(uidx, idxbuf)
    _stream_gather(utabT, utail, p_out, idxbuf, lv, lp, pv, pp, pbuf, rowbuf,
                   wid, psem, osem, tsem)
    pltpu.sync_copy(iidx, idxbuf)
    _stream_gather(itabT, itail, q_out, idxbuf, lv, lp, pv, pp, pbuf, rowbuf,
                   wid, psem, osem, tsem)

    pltpu.sync_copy(rtab, rtv)
    pltpu.sync_copy(ridx.at[pl.ds(base, B_PER_W)], ridxv)

    def rgroup(g, _):
        par = g & 1

        @pl.when(g >= 2)
        def _():
            pltpu.make_async_copy(e_out.at[pl.ds(0, 16)],
                                  rrow.at[par], rsem).wait()

        rv = ridxv[pl.ds(g * 16, 16)]
        rb = rrow.at[par]
        for c in range(EMB_DIM):
            cc = jnp.full((16,), c, jnp.int32)
            col = plsc.load_gather(rtv, [rv, cc])
            plsc.store_scatter(rb, [iota, cc], col)
        pltpu.async_copy(rb, e_out.at[pl.ds(base + g * 16, 16)], rsem)
        return 0

    lax.fori_loop(0, B_PER_W // 16, rgroup, 0)
    pltpu.make_async_copy(e_out.at[pl.ds(0, 16)], rrow.at[0], rsem).wait()
    pltpu.make_async_copy(e_out.at[pl.ds(0, 16)], rrow.at[1], rsem).wait()


@jax.jit
def _run(user_idx, item_idx, rating_idx, utabT, itabT, rtab, utail, itail):
    mesh = plsc.VectorSubcoreMesh(core_axis_name="c", subcore_axis_name="s",
                                  num_cores=NUM_CORES,
                                  num_subcores=NUM_SUBCORES)
    out = jax.ShapeDtypeStruct((BATCH, EMB_DIM), jnp.float32)
    f = pl.kernel(
        _body,
        out_type=(out, out, out),
        mesh=mesh,
        scratch_types=[
            pltpu.VMEM((BATCH,), jnp.int32),
            pltpu.VMEM((B_PER_W,), jnp.int32),
            pltpu.VMEM((5, EMB_DIM), jnp.float32),
            pltpu.VMEM((CAP + 16,), jnp.int32),
            pltpu.VMEM((CAP + 16,), jnp.int32),
            pltpu.VMEM((CAP + 16,), jnp.int32),
            pltpu.VMEM((CAP + 16,), jnp.int32),
            pltpu.VMEM((2, EMB_DIM, PR), jnp.float32),
            pltpu.VMEM((2, 16, EMB_DIM), jnp.float32),
            pltpu.VMEM((2, 16, EMB_DIM), jnp.float32),
            pltpu.SemaphoreType.DMA,
            pltpu.SemaphoreType.DMA,
            pltpu.SemaphoreType.DMA,
            pltpu.SemaphoreType.DMA,
        ],
        compiler_params=pltpu.CompilerParams(needs_layout_passes=False),
    )
    return f(user_idx, item_idx, rating_idx, utabT, itabT, rtab, utail, itail)


def kernel(user_idx, item_idx, rating_idx, user_emb, item_emb, opinion_emb):
    return _run(user_idx.astype(jnp.int32), item_idx.astype(jnp.int32),
                rating_idx.astype(jnp.int32),
                user_emb.T, item_emb.T, opinion_emb,
                user_emb[V_MAIN:], item_emb[V_MAIN:])
